# baseline (speedup 1.0000x reference)
import jax
import jax.numpy as jnp
from jax import lax
from jax.experimental import pallas as pl
from jax.experimental.pallas import tpu as pltpu
from jax.experimental.pallas import tpu_sc as plsc

D = 64
L = 16
R0_CONST = 1.0
K_CONST = 5
B = 16384
NC = 2
NS = 16
NW = NC * NS
BPW = B // NW
CHUNK = 128
NCH = BPW // CHUNK

_MESH = plsc.VectorSubcoreMesh(
    core_axis_name="c", subcore_axis_name="s", num_cores=NC, num_subcores=NS
)


def _sc_body(cidx_hbm, pidx_hbm, nidx_hbm, table_hbm, d2p_hbm, d2n_hbm,
             cidx_v, pidx_v, nidx_v, crows_v, prows_v, nrows_v,
             outp_v, outn_v, sem):
    wid = lax.axis_index("s") * NC + lax.axis_index("c")

    pltpu.sync_copy(cidx_hbm.at[wid], cidx_v)
    pltpu.sync_copy(pidx_hbm.at[wid], pidx_v)
    pltpu.sync_copy(nidx_hbm.at[wid], nidx_v)

    copies = []
    for j in range(NCH):
        sl = pl.ds(j * CHUNK, CHUNK)
        copies.append(pltpu.async_copy(table_hbm.at[cidx_v.at[j]], crows_v.at[sl], sem))
        copies.append(pltpu.async_copy(table_hbm.at[pidx_v.at[j]], prows_v.at[sl], sem))
        copies.append(pltpu.async_copy(table_hbm.at[nidx_v.at[j]], nrows_v.at[sl], sem))
    for cp in copies:
        cp.wait()

    lane = lax.iota(jnp.int32, L)

    def group_body(g, carry):
        rows = g * L + lane
        accp = jnp.zeros((L,), jnp.float32)
        accn = jnp.zeros((L,), jnp.float32)
        for d in range(D):
            col = jnp.full((L,), d, jnp.int32)
            cv = plsc.load_gather(crows_v, [rows, col])
            pv = plsc.load_gather(prows_v, [rows, col])
            nv = plsc.load_gather(nrows_v, [rows, col])
            dp = pv - cv
            dn = nv - cv
            accp = accp + dp * dp
            accn = accn + dn * dn
        outp_v[pl.ds(g * L, L)] = accp
        outn_v[pl.ds(g * L, L)] = accn
        return carry

    lax.fori_loop(0, BPW // L, group_body, 0)

    pltpu.sync_copy(outp_v, d2p_hbm.at[pl.ds(wid * BPW, BPW)])
    pltpu.sync_copy(outn_v, d2n_hbm.at[pl.ds(wid * BPW, BPW)])


_sc_dist2 = pl.kernel(
    _sc_body,
    out_type=(
        jax.ShapeDtypeStruct((B,), jnp.float32),
        jax.ShapeDtypeStruct((B,), jnp.float32),
    ),
    mesh=_MESH,
    scratch_types=[
        pltpu.VMEM((NCH, CHUNK), jnp.int32),
        pltpu.VMEM((NCH, CHUNK), jnp.int32),
        pltpu.VMEM((NCH, CHUNK), jnp.int32),
        pltpu.VMEM((BPW, D), jnp.float32),
        pltpu.VMEM((BPW, D), jnp.float32),
        pltpu.VMEM((BPW, D), jnp.float32),
        pltpu.VMEM((BPW,), jnp.float32),
        pltpu.VMEM((BPW,), jnp.float32),
        pltpu.SemaphoreType.DMA,
    ],
    compiler_params=pltpu.CompilerParams(
        use_tc_tiling_on_sc=False, needs_layout_passes=False
    ),
)


def _loss_body(d2p_ref, d2n_ref, pos_ref, neg_ref):
    dp = jnp.sqrt(d2p_ref[...])
    dn = jnp.sqrt(d2n_ref[...])
    xp = dp - R0_CONST
    xn = R0_CONST - dn
    lp = jnp.sum(jnp.maximum(xp, 0.0) + jnp.log1p(jnp.exp(-jnp.abs(xp))))
    ln = jnp.sum(jnp.maximum(xn, 0.0) + jnp.log1p(jnp.exp(-jnp.abs(xn))))
    pos_ref[0, 0] = lp
    neg_ref[0, 0] = jnp.float32(K_CONST) * ln


_tc_loss = pl.pallas_call(
    _loss_body,
    out_shape=(
        jax.ShapeDtypeStruct((1, 1), jnp.float32),
        jax.ShapeDtypeStruct((1, 1), jnp.float32),
    ),
    out_specs=(
        pl.BlockSpec(memory_space=pltpu.SMEM),
        pl.BlockSpec(memory_space=pltpu.SMEM),
    ),
)


@jax.jit
def kernel(batched_center_word, batched_pos_word, batched_neg_word, graphdist, embed_weight):
    del graphdist
    c = batched_center_word.astype(jnp.int32).reshape(NW, NCH, CHUNK)
    p = batched_pos_word.astype(jnp.int32).reshape(NW, NCH, CHUNK)
    n = batched_neg_word.astype(jnp.int32).reshape(NW, NCH, CHUNK)
    d2p, d2n = _sc_dist2(c, p, n, embed_weight)
    pos, kneg = _tc_loss(d2p.reshape(128, 128), d2n.reshape(128, 128))
    return (pos[0, 0], kneg[0, 0])


# speedup vs baseline: 1.0058x; 1.0058x over previous
import jax
import jax.numpy as jnp
from jax import lax
from jax.experimental import pallas as pl
from jax.experimental.pallas import tpu as pltpu
from jax.experimental.pallas import tpu_sc as plsc

D = 64
L = 16
R0_CONST = 1.0
K_CONST = 5
B = 16384
NC = 2
NS = 16
NW = NC * NS
BPW = B // NW
CHUNK = 128
NCH = BPW // CHUNK
PAIRS = 500000

_MESH = plsc.VectorSubcoreMesh(
    core_axis_name="c", subcore_axis_name="s", num_cores=NC, num_subcores=NS
)


def _sc_body(cidx_hbm, pidx_hbm, nidx_hbm, table_hbm, d2p_hbm, d2n_hbm,
             cidx_v, pidx_v, nidx_v, cpair_v, ppair_v, npair_v,
             gc_v, gp_v, gn_v, outp_v, outn_v, sem):
    wid = lax.axis_index("s") * NC + lax.axis_index("c")

    pltpu.sync_copy(cidx_hbm.at[wid], cidx_v)
    pltpu.sync_copy(pidx_hbm.at[wid], pidx_v)
    pltpu.sync_copy(nidx_hbm.at[wid], nidx_v)

    for j in range(NCH):
        for k in range(CHUNK // L):
            sl = pl.ds(k * L, L)
            cpair_v[j, sl] = lax.shift_right_logical(cidx_v[j, sl], 1)
            ppair_v[j, sl] = lax.shift_right_logical(pidx_v[j, sl], 1)
            npair_v[j, sl] = lax.shift_right_logical(nidx_v[j, sl], 1)

    lane = lax.iota(jnp.int32, L)
    one = jnp.ones((L,), jnp.int32)

    for j in range(NCH):
        cps = [
            pltpu.async_copy(table_hbm.at[cpair_v.at[j]], gc_v, sem),
            pltpu.async_copy(table_hbm.at[ppair_v.at[j]], gp_v, sem),
            pltpu.async_copy(table_hbm.at[npair_v.at[j]], gn_v, sem),
        ]
        for cp in cps:
            cp.wait()

        def group_body(g, carry, j=j):
            rows = g * L + lane
            sl16 = pl.ds(g * L, L)
            chalf = (cidx_v[j, sl16] & one) * 64
            phalf = (pidx_v[j, sl16] & one) * 64
            nhalf = (nidx_v[j, sl16] & one) * 64
            accp = jnp.zeros((L,), jnp.float32)
            accn = jnp.zeros((L,), jnp.float32)
            for d in range(D):
                cv = plsc.load_gather(gc_v, [rows, chalf + d])
                pv = plsc.load_gather(gp_v, [rows, phalf + d])
                nv = plsc.load_gather(gn_v, [rows, nhalf + d])
                dp = pv - cv
                dn = nv - cv
                accp = accp + dp * dp
                accn = accn + dn * dn
            outp_v[pl.ds(j * CHUNK + g * L, L)] = accp
            outn_v[pl.ds(j * CHUNK + g * L, L)] = accn
            return carry

        lax.fori_loop(0, CHUNK // L, group_body, 0)

    pltpu.sync_copy(outp_v, d2p_hbm.at[pl.ds(wid * BPW, BPW)])
    pltpu.sync_copy(outn_v, d2n_hbm.at[pl.ds(wid * BPW, BPW)])


_sc_dist2 = pl.kernel(
    _sc_body,
    out_type=(
        jax.ShapeDtypeStruct((B,), jnp.float32),
        jax.ShapeDtypeStruct((B,), jnp.float32),
    ),
    mesh=_MESH,
    scratch_types=[
        pltpu.VMEM((NCH, CHUNK), jnp.int32),
        pltpu.VMEM((NCH, CHUNK), jnp.int32),
        pltpu.VMEM((NCH, CHUNK), jnp.int32),
        pltpu.VMEM((NCH, CHUNK), jnp.int32),
        pltpu.VMEM((NCH, CHUNK), jnp.int32),
        pltpu.VMEM((NCH, CHUNK), jnp.int32),
        pltpu.VMEM((CHUNK, 2 * D), jnp.float32),
        pltpu.VMEM((CHUNK, 2 * D), jnp.float32),
        pltpu.VMEM((CHUNK, 2 * D), jnp.float32),
        pltpu.VMEM((BPW,), jnp.float32),
        pltpu.VMEM((BPW,), jnp.float32),
        pltpu.SemaphoreType.DMA,
    ],
    compiler_params=pltpu.CompilerParams(needs_layout_passes=False),
)


def _loss_body(d2p_ref, d2n_ref, pos_ref, neg_ref):
    dp = jnp.sqrt(d2p_ref[...])
    dn = jnp.sqrt(d2n_ref[...])
    xp = dp - R0_CONST
    xn = R0_CONST - dn
    lp = jnp.sum(jnp.maximum(xp, 0.0) + jnp.log1p(jnp.exp(-jnp.abs(xp))))
    ln = jnp.sum(jnp.maximum(xn, 0.0) + jnp.log1p(jnp.exp(-jnp.abs(xn))))
    pos_ref[0, 0] = lp
    neg_ref[0, 0] = jnp.float32(K_CONST) * ln


_tc_loss = pl.pallas_call(
    _loss_body,
    out_shape=(
        jax.ShapeDtypeStruct((1, 1), jnp.float32),
        jax.ShapeDtypeStruct((1, 1), jnp.float32),
    ),
    out_specs=(
        pl.BlockSpec(memory_space=pltpu.SMEM),
        pl.BlockSpec(memory_space=pltpu.SMEM),
    ),
)


@jax.jit
def kernel(batched_center_word, batched_pos_word, batched_neg_word, graphdist, embed_weight):
    del graphdist
    c = batched_center_word.astype(jnp.int32).reshape(NW, NCH, CHUNK)
    p = batched_pos_word.astype(jnp.int32).reshape(NW, NCH, CHUNK)
    n = batched_neg_word.astype(jnp.int32).reshape(NW, NCH, CHUNK)
    table2 = embed_weight.reshape(PAIRS, 2 * D)
    d2p, d2n = _sc_dist2(c, p, n, table2)
    pos, kneg = _tc_loss(d2p.reshape(128, 128), d2n.reshape(128, 128))
    return (pos[0, 0], kneg[0, 0])


# speedup vs baseline: 1.8661x; 1.8554x over previous
import jax
import jax.numpy as jnp
from jax import lax
from jax.experimental import pallas as pl
from jax.experimental.pallas import tpu as pltpu
from jax.experimental.pallas import tpu_sc as plsc

D = 64
L = 16
R0_CONST = 1.0
K_CONST = 5
B = 16384
NC = 2
NS = 16
NW = NC * NS
BPW = B // NW
CHUNK = 32
NCH = BPW // CHUNK
NBLK = 125000

_MESH = plsc.VectorSubcoreMesh(
    core_axis_name="c", subcore_axis_name="s", num_cores=NC, num_subcores=NS
)


def _sc_body(cidx_hbm, pidx_hbm, nidx_hbm, table_hbm, d2p_hbm, d2n_hbm,
             cidx_v, pidx_v, nidx_v,
             gc_v, gp_v, gn_v, outp_v, outn_v, sem):
    wid = lax.axis_index("s") * NC + lax.axis_index("c")

    pltpu.sync_copy(cidx_hbm.at[wid], cidx_v)
    pltpu.sync_copy(pidx_hbm.at[wid], pidx_v)
    pltpu.sync_copy(nidx_hbm.at[wid], nidx_v)

    lane = lax.iota(jnp.int32, L)
    seven = jnp.full((L,), 7, jnp.int32)

    for j in range(NCH):
        def fetch_body(g, carry, j=j):
            sl16 = pl.ds(g * L, L)
            bc = lax.shift_right_logical(cidx_v[j, sl16], 3)
            bp = lax.shift_right_logical(pidx_v[j, sl16], 3)
            bn = lax.shift_right_logical(nidx_v[j, sl16], 3)
            for k in range(L):
                i = g * L + k
                pltpu.async_copy(table_hbm.at[bc[k]], gc_v.at[i], sem)
                pltpu.async_copy(table_hbm.at[bp[k]], gp_v.at[i], sem)
                pltpu.async_copy(table_hbm.at[bn[k]], gn_v.at[i], sem)
            return carry

        lax.fori_loop(0, CHUNK // L, fetch_body, 0)

        def drain_body(i, carry):
            pltpu.make_async_copy(table_hbm.at[0], gc_v.at[i], sem).wait()
            pltpu.make_async_copy(table_hbm.at[0], gp_v.at[i], sem).wait()
            pltpu.make_async_copy(table_hbm.at[0], gn_v.at[i], sem).wait()
            return carry

        lax.fori_loop(0, CHUNK, drain_body, 0)

        def group_body(g, carry, j=j):
            rows = g * L + lane
            sl16 = pl.ds(g * L, L)
            csub = cidx_v[j, sl16] & seven
            psub = pidx_v[j, sl16] & seven
            nsub = nidx_v[j, sl16] & seven
            accp = jnp.zeros((L,), jnp.float32)
            accn = jnp.zeros((L,), jnp.float32)
            for d in range(D):
                col = jnp.full((L,), d, jnp.int32)
                cv = plsc.load_gather(gc_v, [rows, csub, col])
                pv = plsc.load_gather(gp_v, [rows, psub, col])
                nv = plsc.load_gather(gn_v, [rows, nsub, col])
                dp = pv - cv
                dn = nv - cv
                accp = accp + dp * dp
                accn = accn + dn * dn
            outp_v[pl.ds(j * CHUNK + g * L, L)] = accp
            outn_v[pl.ds(j * CHUNK + g * L, L)] = accn
            return carry

        lax.fori_loop(0, CHUNK // L, group_body, 0)

    pltpu.sync_copy(outp_v, d2p_hbm.at[pl.ds(wid * BPW, BPW)])
    pltpu.sync_copy(outn_v, d2n_hbm.at[pl.ds(wid * BPW, BPW)])


_sc_dist2 = pl.kernel(
    _sc_body,
    out_type=(
        jax.ShapeDtypeStruct((B,), jnp.float32),
        jax.ShapeDtypeStruct((B,), jnp.float32),
    ),
    mesh=_MESH,
    scratch_types=[
        pltpu.VMEM((NCH, CHUNK), jnp.int32),
        pltpu.VMEM((NCH, CHUNK), jnp.int32),
        pltpu.VMEM((NCH, CHUNK), jnp.int32),
        pltpu.VMEM((CHUNK, 8, D), jnp.float32),
        pltpu.VMEM((CHUNK, 8, D), jnp.float32),
        pltpu.VMEM((CHUNK, 8, D), jnp.float32),
        pltpu.VMEM((BPW,), jnp.float32),
        pltpu.VMEM((BPW,), jnp.float32),
        pltpu.SemaphoreType.DMA,
    ],
    compiler_params=pltpu.CompilerParams(needs_layout_passes=False),
)


def _loss_body(d2p_ref, d2n_ref, pos_ref, neg_ref):
    dp = jnp.sqrt(d2p_ref[...])
    dn = jnp.sqrt(d2n_ref[...])
    xp = dp - R0_CONST
    xn = R0_CONST - dn
    lp = jnp.sum(jnp.maximum(xp, 0.0) + jnp.log1p(jnp.exp(-jnp.abs(xp))))
    ln = jnp.sum(jnp.maximum(xn, 0.0) + jnp.log1p(jnp.exp(-jnp.abs(xn))))
    pos_ref[0, 0] = lp
    neg_ref[0, 0] = jnp.float32(K_CONST) * ln


_tc_loss = pl.pallas_call(
    _loss_body,
    out_shape=(
        jax.ShapeDtypeStruct((1, 1), jnp.float32),
        jax.ShapeDtypeStruct((1, 1), jnp.float32),
    ),
    out_specs=(
        pl.BlockSpec(memory_space=pltpu.SMEM),
        pl.BlockSpec(memory_space=pltpu.SMEM),
    ),
)


@jax.jit
def kernel(batched_center_word, batched_pos_word, batched_neg_word, graphdist, embed_weight):
    del graphdist
    c = batched_center_word.astype(jnp.int32).reshape(NW, NCH, CHUNK)
    p = batched_pos_word.astype(jnp.int32).reshape(NW, NCH, CHUNK)
    n = batched_neg_word.astype(jnp.int32).reshape(NW, NCH, CHUNK)
    table3 = embed_weight.reshape(NBLK, 8, D)
    d2p, d2n = _sc_dist2(c, p, n, table3)
    pos, kneg = _tc_loss(d2p.reshape(128, 128), d2n.reshape(128, 128))
    return (pos[0, 0], kneg[0, 0])


# speedup vs baseline: 2.1597x; 1.1573x over previous
import jax
import jax.numpy as jnp
from jax import lax
from jax.experimental import pallas as pl
from jax.experimental.pallas import tpu as pltpu
from jax.experimental.pallas import tpu_sc as plsc

D = 64
L = 16
R0_CONST = 1.0
K_CONST = 5
B = 16384
NC = 2
NS = 16
NW = NC * NS
BPW = B // NW
CHUNK = 16
NCH = BPW // CHUNK
NBLK = 125000

_MESH = plsc.VectorSubcoreMesh(
    core_axis_name="c", subcore_axis_name="s", num_cores=NC, num_subcores=NS
)


def _sc_body(cidx_hbm, pidx_hbm, nidx_hbm, table_hbm, d2p_hbm, d2n_hbm,
             cidx_v, pidx_v, nidx_v,
             gc0_v, gp0_v, gn0_v, gc1_v, gp1_v, gn1_v,
             outp_v, outn_v, sem0, sem1):
    wid = lax.axis_index("s") * NC + lax.axis_index("c")

    pltpu.sync_copy(cidx_hbm.at[wid], cidx_v)
    pltpu.sync_copy(pidx_hbm.at[wid], pidx_v)
    pltpu.sync_copy(nidx_hbm.at[wid], nidx_v)

    lane = lax.iota(jnp.int32, L)
    seven = jnp.full((L,), 7, jnp.int32)
    sl16 = pl.ds(0, L)

    def fetch(c, bufs, sem):
        gc_v, gp_v, gn_v = bufs
        bc = lax.shift_right_logical(cidx_v[c, sl16], 3)
        bp = lax.shift_right_logical(pidx_v[c, sl16], 3)
        bn = lax.shift_right_logical(nidx_v[c, sl16], 3)
        for k in range(L):
            pltpu.async_copy(table_hbm.at[bc[k]], gc_v.at[k], sem)
            pltpu.async_copy(table_hbm.at[bp[k]], gp_v.at[k], sem)
            pltpu.async_copy(table_hbm.at[bn[k]], gn_v.at[k], sem)

    def drain(bufs, sem):
        gc_v, gp_v, gn_v = bufs
        for k in range(L):
            pltpu.make_async_copy(table_hbm.at[0], gc_v.at[k], sem).wait()
            pltpu.make_async_copy(table_hbm.at[0], gp_v.at[k], sem).wait()
            pltpu.make_async_copy(table_hbm.at[0], gn_v.at[k], sem).wait()

    def compute(c, bufs):
        gc_v, gp_v, gn_v = bufs
        csub = cidx_v[c, sl16] & seven
        psub = pidx_v[c, sl16] & seven
        nsub = nidx_v[c, sl16] & seven
        accp = jnp.zeros((L,), jnp.float32)
        accn = jnp.zeros((L,), jnp.float32)
        for d in range(D):
            col = jnp.full((L,), d, jnp.int32)
            cv = plsc.load_gather(gc_v, [lane, csub, col])
            pv = plsc.load_gather(gp_v, [lane, psub, col])
            nv = plsc.load_gather(gn_v, [lane, nsub, col])
            dp = pv - cv
            dn = nv - cv
            accp = accp + dp * dp
            accn = accn + dn * dn
        outp_v[pl.ds(c * L, L)] = accp
        outn_v[pl.ds(c * L, L)] = accn

    bufs_a = (gc0_v, gp0_v, gn0_v)
    bufs_b = (gc1_v, gp1_v, gn1_v)

    fetch(0, bufs_a, sem0)

    def pipe_body(t, carry):
        fetch(2 * t + 1, bufs_b, sem1)
        drain(bufs_a, sem0)
        compute(2 * t, bufs_a)
        fetch(lax.rem(2 * t + 2, NCH), bufs_a, sem0)
        drain(bufs_b, sem1)
        compute(2 * t + 1, bufs_b)
        return carry

    lax.fori_loop(0, NCH // 2, pipe_body, 0)
    drain(bufs_a, sem0)

    pltpu.sync_copy(outp_v, d2p_hbm.at[pl.ds(wid * BPW, BPW)])
    pltpu.sync_copy(outn_v, d2n_hbm.at[pl.ds(wid * BPW, BPW)])


_sc_dist2 = pl.kernel(
    _sc_body,
    out_type=(
        jax.ShapeDtypeStruct((B,), jnp.float32),
        jax.ShapeDtypeStruct((B,), jnp.float32),
    ),
    mesh=_MESH,
    scratch_types=[
        pltpu.VMEM((NCH, CHUNK), jnp.int32),
        pltpu.VMEM((NCH, CHUNK), jnp.int32),
        pltpu.VMEM((NCH, CHUNK), jnp.int32),
        pltpu.VMEM((CHUNK, 8, D), jnp.float32),
        pltpu.VMEM((CHUNK, 8, D), jnp.float32),
        pltpu.VMEM((CHUNK, 8, D), jnp.float32),
        pltpu.VMEM((CHUNK, 8, D), jnp.float32),
        pltpu.VMEM((CHUNK, 8, D), jnp.float32),
        pltpu.VMEM((CHUNK, 8, D), jnp.float32),
        pltpu.VMEM((BPW,), jnp.float32),
        pltpu.VMEM((BPW,), jnp.float32),
        pltpu.SemaphoreType.DMA,
        pltpu.SemaphoreType.DMA,
    ],
    compiler_params=pltpu.CompilerParams(needs_layout_passes=False),
)


def _loss_body(d2p_ref, d2n_ref, pos_ref, neg_ref):
    dp = jnp.sqrt(d2p_ref[...])
    dn = jnp.sqrt(d2n_ref[...])
    xp = dp - R0_CONST
    xn = R0_CONST - dn
    lp = jnp.sum(jnp.maximum(xp, 0.0) + jnp.log1p(jnp.exp(-jnp.abs(xp))))
    ln = jnp.sum(jnp.maximum(xn, 0.0) + jnp.log1p(jnp.exp(-jnp.abs(xn))))
    pos_ref[0, 0] = lp
    neg_ref[0, 0] = jnp.float32(K_CONST) * ln


_tc_loss = pl.pallas_call(
    _loss_body,
    out_shape=(
        jax.ShapeDtypeStruct((1, 1), jnp.float32),
        jax.ShapeDtypeStruct((1, 1), jnp.float32),
    ),
    out_specs=(
        pl.BlockSpec(memory_space=pltpu.SMEM),
        pl.BlockSpec(memory_space=pltpu.SMEM),
    ),
)


@jax.jit
def kernel(batched_center_word, batched_pos_word, batched_neg_word, graphdist, embed_weight):
    del graphdist
    c = batched_center_word.astype(jnp.int32).reshape(NW, NCH, CHUNK)
    p = batched_pos_word.astype(jnp.int32).reshape(NW, NCH, CHUNK)
    n = batched_neg_word.astype(jnp.int32).reshape(NW, NCH, CHUNK)
    table3 = embed_weight.reshape(NBLK, 8, D)
    d2p, d2n = _sc_dist2(c, p, n, table3)
    pos, kneg = _tc_loss(d2p.reshape(128, 128), d2n.reshape(128, 128))
    return (pos[0, 0], kneg[0, 0])
